# baseline (device time: 55341 ns/iter reference)
import jax
import jax.numpy as jnp
from jax import lax
from jax.experimental import pallas as pl
from jax.experimental.pallas import tpu as pltpu

N_DEV = 8
B = 2
SQ_LOC = 128
D_MODEL = 512
HQ = 32
DH = 64
D_FF = HQ * DH
CHUNK = D_FF // N_DEV
HG = HQ // N_DEV
SKV = 128
CW_HOPS = 4
CCW_HOPS = 3
CW_SRC = (0, 1, 3, 5)
CW_DST = (1, 3, 5, 7)
CCW_SRC = (0, 2, 4)
CCW_DST = (2, 4, 6)


def _f(t):
    return jnp.where(t < 4, t, 11 - t)


def kernel(x, Wq, K_ext, V_ext, Wo):
    def body(x_ref, wq_ref, k_ref, v_ref, wo_ref, out_ref,
             wq_full, wo_full, xb, k_cm, v_cm, q_cm, k_pr, v_pr, ctx_g, acc,
             cw_send, cw_recv, ccw_send, ccw_recv):
        me = lax.axis_index("i")
        cp = _f(me)
        nxt = _f(jnp.mod(cp + 1, N_DEV))
        prv = _f(jnp.mod(cp - 1, N_DEV))
        is_even = jnp.mod(me, 2) == 0

        def origin(slot):
            if slot == 0:
                return me
            k = (slot + 1) // 2
            return _f(jnp.mod(cp - k, N_DEV) if slot % 2 else
                      jnp.mod(cp + k, N_DEV))

        wq_full[:, 0:CHUNK] = wq_ref[...].astype(jnp.bfloat16)
        wo_full[0:CHUNK, :] = wo_ref[...].astype(jnp.bfloat16)

        barrier = pltpu.get_barrier_semaphore()
        for nbr in (nxt, prv):
            pl.semaphore_signal(
                barrier, inc=1, device_id=(nbr,),
                device_id_type=pl.DeviceIdType.MESH,
            )
        pl.semaphore_wait(barrier, 2)

        def start_pair(src, dst, sems_s, sems_r, h, target):
            rq = pltpu.make_async_remote_copy(
                src_ref=wq_full.at[:, pl.ds(src * CHUNK, CHUNK)],
                dst_ref=wq_full.at[:, pl.ds(dst * CHUNK, CHUNK)],
                send_sem=sems_s.at[0, h],
                recv_sem=sems_r.at[0, h],
                device_id=(target,),
                device_id_type=pl.DeviceIdType.MESH,
            )
            ro = pltpu.make_async_remote_copy(
                src_ref=wo_full.at[pl.ds(src * CHUNK, CHUNK), :],
                dst_ref=wo_full.at[pl.ds(dst * CHUNK, CHUNK), :],
                send_sem=sems_s.at[1, h],
                recv_sem=sems_r.at[1, h],
                device_id=(target,),
                device_id_type=pl.DeviceIdType.MESH,
            )
            rq.start()
            ro.start()
            return rq, ro

        def compute_group(slot0, nslots, first):
            c0 = slot0 * CHUNK
            nc = nslots * CHUNK
            nbh = nslots * B * HG
            q_g = jnp.dot(xb[...], wq_full[:, c0:c0 + nc],
                          preferred_element_type=jnp.float32)
            q_g = q_g.astype(jnp.bfloat16)
            for si in range(nslots):
                o = origin(slot0 + si)
                k_pr[si * B * HG:(si + 1) * B * HG] = k_cm[pl.ds(o, 1)].reshape(
                    B * HG, SKV, DH)
                v_pr[si * B * HG:(si + 1) * B * HG] = v_cm[pl.ds(o, 1)].reshape(
                    B * HG, SKV, DH)
                for b in range(B):
                    for hh in range(HG):
                        q_cm[si * B * HG + b * HG + hh] = q_g[
                            b * SQ_LOC:(b + 1) * SQ_LOC,
                            si * CHUNK + hh * DH:si * CHUNK + (hh + 1) * DH]
            qv = q_cm[0:nbh].reshape(nbh * 2, 64, DH)
            kv = k_pr[0:nbh].reshape(nbh * 2, 64, DH)
            s = lax.dot_general(
                qv, kv,
                dimension_numbers=(((2,), (2,)), ((0,), (0,))),
                preferred_element_type=jnp.float32,
            ) * 0.125
            m = jnp.max(s, axis=-1, keepdims=True)
            w = jnp.exp(s - m)
            wsum = jnp.sum(w, axis=-1, keepdims=True)
            w = (w / wsum).astype(jnp.bfloat16)
            ctx = lax.dot_general(
                w, v_pr[0:nbh].reshape(nbh * 2, 64, DH),
                dimension_numbers=(((2,), (1,)), ((0,), (0,))),
                preferred_element_type=jnp.float32,
            ).reshape(nbh, SQ_LOC, DH)
            for si in range(nslots):
                for b in range(B):
                    for hh in range(HG):
                        ctx_g[b * SQ_LOC:(b + 1) * SQ_LOC,
                              si * CHUNK + hh * DH:si * CHUNK + (hh + 1) * DH] \
                            = ctx[si * B * HG + b * HG + hh].astype(jnp.bfloat16)
            part = jnp.dot(ctx_g[...][:, 0:nc], wo_full[c0:c0 + nc, :],
                           preferred_element_type=jnp.float32)
            if first:
                acc[...] = part
            else:
                acc[...] = acc[...] + part

        cw = start_pair(CW_SRC[0], CW_DST[0], cw_send, cw_recv, 0, nxt)
        ccw = start_pair(CCW_SRC[0], CCW_DST[0], ccw_send, ccw_recv, 0, prv)

        @pl.when(is_even)
        def _():
            xb[...] = x_ref[...].reshape(B * SQ_LOC, D_MODEL).astype(
                jnp.bfloat16)
            for g in range(N_DEV):
                for b in range(B):
                    for hh in range(HG):
                        k_cm[g, b * HG + hh] = k_ref[
                            b, :, g * HG + hh, :].astype(jnp.bfloat16)
                        v_cm[g, b * HG + hh] = v_ref[
                            b, :, g * HG + hh, :].astype(jnp.bfloat16)
            compute_group(0, 1, first=True)

        for r in cw + ccw:
            r.wait()

        for h in range(1, CW_HOPS):
            cw = start_pair(CW_SRC[h], CW_DST[h], cw_send, cw_recv, h, nxt)
            ccw = (start_pair(CCW_SRC[h], CCW_DST[h], ccw_send, ccw_recv,
                              h, prv) if h < CCW_HOPS else None)

            @pl.when(is_even)
            def _(h=h):
                compute_group(2 * h - 1, 2, first=False)

            for r in cw:
                r.wait()
            if ccw is not None:
                for r in ccw:
                    r.wait()

        @pl.when(is_even)
        def _():
            compute_group(7, 1, first=False)
            out_ref[...] = acc[...].reshape(B, SQ_LOC, D_MODEL)

        @pl.when(jnp.logical_not(is_even))
        def _():
            out_ref[...] = jnp.zeros((B, SQ_LOC, D_MODEL), jnp.float32)

    return pl.pallas_call(
        body,
        out_shape=jax.ShapeDtypeStruct((B, SQ_LOC, D_MODEL), jnp.float32),
        in_specs=[pl.BlockSpec(memory_space=pltpu.VMEM)] * 5,
        out_specs=pl.BlockSpec(memory_space=pltpu.VMEM),
        scratch_shapes=[
            pltpu.VMEM((D_MODEL, D_FF), jnp.bfloat16),
            pltpu.VMEM((D_FF, D_MODEL), jnp.bfloat16),
            pltpu.VMEM((B * SQ_LOC, D_MODEL), jnp.bfloat16),
            pltpu.VMEM((N_DEV, B * HG, SKV, DH), jnp.bfloat16),
            pltpu.VMEM((N_DEV, B * HG, SKV, DH), jnp.bfloat16),
            pltpu.VMEM((2 * B * HG, SQ_LOC, DH), jnp.bfloat16),
            pltpu.VMEM((2 * B * HG, SKV, DH), jnp.bfloat16),
            pltpu.VMEM((2 * B * HG, SKV, DH), jnp.bfloat16),
            pltpu.VMEM((B * SQ_LOC, 2 * CHUNK), jnp.bfloat16),
            pltpu.VMEM((B * SQ_LOC, D_MODEL), jnp.float32),
            pltpu.SemaphoreType.DMA((2, CW_HOPS)),
            pltpu.SemaphoreType.DMA((2, CW_HOPS)),
            pltpu.SemaphoreType.DMA((2, CCW_HOPS)),
            pltpu.SemaphoreType.DMA((2, CCW_HOPS)),
        ],
        compiler_params=pltpu.CompilerParams(collective_id=0),
    )(x, Wq, K_ext, V_ext, Wo)


# device time: 41169 ns/iter; 1.3442x vs baseline; 1.3442x over previous
import jax
import jax.numpy as jnp
from jax import lax
from jax.experimental import pallas as pl
from jax.experimental.pallas import tpu as pltpu

N_DEV = 8
B = 2
SQ_LOC = 128
D_MODEL = 512
HQ = 32
DH = 64
D_FF = HQ * DH
CHUNK = D_FF // N_DEV
HG = HQ // N_DEV
SKV = 128

MASKS = (1, 3, 4)
SLOT_XOR = (0, 1, 3, 4, 2, 7, 5, 6)


def kernel(x, Wq, K_ext, V_ext, Wo):
    def body(x_ref, wq_ref, k_ref, v_ref, wo_ref, out_ref,
             wq_full, wo_full, xb, k_cm, v_cm, k_ord, v_ord, q_hm, ctx_buf,
             p1_send, p1_recv, p2_send, p2_recv, p3_send, p3_recv):
        me = lax.axis_index("i")
        is_even = jnp.mod(me, 2) == 0
        nbr = [jnp.bitwise_xor(me, m) for m in MASKS]

        wq_full[:, 0:CHUNK] = wq_ref[...].astype(jnp.bfloat16)
        wo_full[0:CHUNK, :] = wo_ref[...].astype(jnp.bfloat16)

        barrier = pltpu.get_barrier_semaphore()
        for n in nbr:
            pl.semaphore_signal(
                barrier, inc=1, device_id=(n,),
                device_id_type=pl.DeviceIdType.MESH,
            )
        pl.semaphore_wait(barrier, 3)

        def chunk_copy(src, dst, sems_s, sems_r, i, target):
            rq = pltpu.make_async_remote_copy(
                src_ref=wq_full.at[:, pl.ds(src * CHUNK, CHUNK)],
                dst_ref=wq_full.at[:, pl.ds(dst * CHUNK, CHUNK)],
                send_sem=sems_s.at[0, i],
                recv_sem=sems_r.at[0, i],
                device_id=(target,),
                device_id_type=pl.DeviceIdType.MESH,
            )
            ro = pltpu.make_async_remote_copy(
                src_ref=wo_full.at[pl.ds(src * CHUNK, CHUNK), :],
                dst_ref=wo_full.at[pl.ds(dst * CHUNK, CHUNK), :],
                send_sem=sems_s.at[1, i],
                recv_sem=sems_r.at[1, i],
                device_id=(target,),
                device_id_type=pl.DeviceIdType.MESH,
            )
            rq.start()
            ro.start()
            return [rq, ro]

        p1 = []
        for i in range(3):
            p1 += chunk_copy(0, 1 + i, p1_send, p1_recv, i, nbr[i])

        @pl.when(is_even)
        def _():
            xb[...] = x_ref[...].reshape(B * SQ_LOC, D_MODEL).astype(
                jnp.bfloat16)
            for g in range(N_DEV):
                for b in range(B):
                    for hh in range(HG):
                        k_cm[g, b * HG + hh] = k_ref[
                            b, :, g * HG + hh, :].astype(jnp.bfloat16)
                        v_cm[g, b * HG + hh] = v_ref[
                            b, :, g * HG + hh, :].astype(jnp.bfloat16)

        for r in p1:
            r.wait()

        p2 = []
        for i in range(3):
            p2 += chunk_copy(1 + (i + 1) % 3, 4 + i, p2_send, p2_recv,
                             i, nbr[i])
        for r in p2:
            r.wait()

        HALF = CHUNK // 2
        p3 = []
        rq = pltpu.make_async_remote_copy(
            src_ref=wq_full.at[:, pl.ds(5 * CHUNK, HALF)],
            dst_ref=wq_full.at[:, pl.ds(7 * CHUNK, HALF)],
            send_sem=p3_send.at[0], recv_sem=p3_recv.at[0],
            device_id=(nbr[0],), device_id_type=pl.DeviceIdType.MESH,
        )
        rq.start()
        p3.append(rq)
        rq = pltpu.make_async_remote_copy(
            src_ref=wq_full.at[:, pl.ds(6 * CHUNK + HALF, HALF)],
            dst_ref=wq_full.at[:, pl.ds(7 * CHUNK + HALF, HALF)],
            send_sem=p3_send.at[1], recv_sem=p3_recv.at[1],
            device_id=(nbr[1],), device_id_type=pl.DeviceIdType.MESH,
        )
        rq.start()
        p3.append(rq)
        rq = pltpu.make_async_remote_copy(
            src_ref=wo_full.at[pl.ds(4 * CHUNK, CHUNK), :],
            dst_ref=wo_full.at[pl.ds(7 * CHUNK, CHUNK), :],
            send_sem=p3_send.at[2], recv_sem=p3_recv.at[2],
            device_id=(nbr[2],), device_id_type=pl.DeviceIdType.MESH,
        )
        rq.start()
        p3.append(rq)
        for r in p3:
            r.wait()

        @pl.when(is_even)
        def _():
            for s in range(N_DEV):
                o = jnp.bitwise_xor(me, SLOT_XOR[s])
                k_ord[s * B * HG:(s + 1) * B * HG] = k_cm[pl.ds(o, 1)].reshape(
                    B * HG, SKV, DH)
                v_ord[s * B * HG:(s + 1) * B * HG] = v_cm[pl.ds(o, 1)].reshape(
                    B * HG, SKV, DH)

            q2d = jnp.dot(xb[...], wq_full[...],
                          preferred_element_type=jnp.float32)
            q4 = q2d.reshape(B, SQ_LOC, HQ, DH).astype(jnp.bfloat16)
            for s in range(N_DEV):
                for b in range(B):
                    for hh in range(HG):
                        q_hm[s * B * HG + b * HG + hh] = q4[b, :, s * HG + hh, :]

            qv = q_hm[...].reshape(N_DEV * B * HG * 2, 64, DH)
            kv = k_ord[...].reshape(N_DEV * B * HG * 2, 64, DH)
            s_ = lax.dot_general(
                qv, kv,
                dimension_numbers=(((2,), (2,)), ((0,), (0,))),
                preferred_element_type=jnp.float32,
            ) * 0.125
            m = jnp.max(s_, axis=-1, keepdims=True)
            w = jnp.exp(s_ - m)
            wsum = jnp.sum(w, axis=-1, keepdims=True)
            w = (w / wsum).astype(jnp.bfloat16)
            ctx = lax.dot_general(
                w, v_ord[...].reshape(N_DEV * B * HG * 2, 64, DH),
                dimension_numbers=(((2,), (1,)), ((0,), (0,))),
                preferred_element_type=jnp.float32,
            ).reshape(N_DEV * B * HG, SQ_LOC, DH)
            for s in range(N_DEV):
                for b in range(B):
                    for hh in range(HG):
                        col = (s * HG + hh) * DH
                        ctx_buf[pl.ds(b * SQ_LOC, SQ_LOC),
                                pl.ds(col, DH)] = (
                            ctx[s * B * HG + b * HG + hh].astype(jnp.bfloat16))

            out2d = jnp.dot(ctx_buf[...], wo_full[...],
                            preferred_element_type=jnp.float32)
            out_ref[...] = out2d.reshape(B, SQ_LOC, D_MODEL)

        @pl.when(jnp.logical_not(is_even))
        def _():
            out_ref[...] = jnp.zeros((B, SQ_LOC, D_MODEL), jnp.float32)

    return pl.pallas_call(
        body,
        out_shape=jax.ShapeDtypeStruct((B, SQ_LOC, D_MODEL), jnp.float32),
        in_specs=[pl.BlockSpec(memory_space=pltpu.VMEM)] * 5,
        out_specs=pl.BlockSpec(memory_space=pltpu.VMEM),
        scratch_shapes=[
            pltpu.VMEM((D_MODEL, D_FF), jnp.bfloat16),
            pltpu.VMEM((D_FF, D_MODEL), jnp.bfloat16),
            pltpu.VMEM((B * SQ_LOC, D_MODEL), jnp.bfloat16),
            pltpu.VMEM((N_DEV, B * HG, SKV, DH), jnp.bfloat16),
            pltpu.VMEM((N_DEV, B * HG, SKV, DH), jnp.bfloat16),
            pltpu.VMEM((N_DEV * B * HG, SKV, DH), jnp.bfloat16),
            pltpu.VMEM((N_DEV * B * HG, SKV, DH), jnp.bfloat16),
            pltpu.VMEM((N_DEV * B * HG, SQ_LOC, DH), jnp.bfloat16),
            pltpu.VMEM((B * SQ_LOC, D_FF), jnp.bfloat16),
            pltpu.SemaphoreType.DMA((2, 3)),
            pltpu.SemaphoreType.DMA((2, 3)),
            pltpu.SemaphoreType.DMA((2, 3)),
            pltpu.SemaphoreType.DMA((2, 3)),
            pltpu.SemaphoreType.DMA((3,)),
            pltpu.SemaphoreType.DMA((3,)),
        ],
        compiler_params=pltpu.CompilerParams(collective_id=0),
    )(x, Wq, K_ext, V_ext, Wo)


# device time: 36030 ns/iter; 1.5360x vs baseline; 1.1426x over previous
import jax
import jax.numpy as jnp
from jax import lax
from jax.experimental import pallas as pl
from jax.experimental.pallas import tpu as pltpu

N_DEV = 8
B = 2
SQ_LOC = 128
D_MODEL = 512
HQ = 32
DH = 64
D_FF = HQ * DH
CHUNK = D_FF // N_DEV
HG = HQ // N_DEV
SKV = 128

MASKS = (1, 3, 4)
SLOT_XOR = (0, 1, 3, 4, 2, 7, 5, 6)
HALF = CHUNK // 2
WO_CUT = (0, 48, 96, CHUNK)


def kernel(x, Wq, K_ext, V_ext, Wo):
    def body(x_ref, wq_ref, k_ref, v_ref, wo_ref, out_ref,
             wq_full, wo_full, xb, k_cm, v_cm, k_ord, v_ord, q_hm, ctx_buf,
             acc, p1_send, p1_recv, p2_send, p2_recv, p3_send, p3_recv):
        me = lax.axis_index("i")
        is_even = jnp.mod(me, 2) == 0
        nbr = [jnp.bitwise_xor(me, m) for m in MASKS]

        wq_full[:, 0:CHUNK] = wq_ref[...].astype(jnp.bfloat16)
        wo_full[0:CHUNK, :] = wo_ref[...].astype(jnp.bfloat16)

        barrier = pltpu.get_barrier_semaphore()
        for n in nbr:
            pl.semaphore_signal(
                barrier, inc=1, device_id=(n,),
                device_id_type=pl.DeviceIdType.MESH,
            )
        pl.semaphore_wait(barrier, 3)

        def chunk_copy(src, dst, sems_s, sems_r, i, target):
            rq = pltpu.make_async_remote_copy(
                src_ref=wq_full.at[:, pl.ds(src * CHUNK, CHUNK)],
                dst_ref=wq_full.at[:, pl.ds(dst * CHUNK, CHUNK)],
                send_sem=sems_s.at[0, i],
                recv_sem=sems_r.at[0, i],
                device_id=(target,),
                device_id_type=pl.DeviceIdType.MESH,
            )
            ro = pltpu.make_async_remote_copy(
                src_ref=wo_full.at[pl.ds(src * CHUNK, CHUNK), :],
                dst_ref=wo_full.at[pl.ds(dst * CHUNK, CHUNK), :],
                send_sem=sems_s.at[1, i],
                recv_sem=sems_r.at[1, i],
                device_id=(target,),
                device_id_type=pl.DeviceIdType.MESH,
            )
            rq.start()
            ro.start()
            return [rq, ro]

        def compute_part(slot0, nslots, first):
            c0 = slot0 * CHUNK
            nc = nslots * CHUNK
            nbh = nslots * B * HG
            i0 = slot0 * B * HG
            q_g = jnp.dot(xb[...], wq_full[:, c0:c0 + nc],
                          preferred_element_type=jnp.float32)
            q4 = q_g.reshape(B, SQ_LOC, nslots * HG, DH).astype(jnp.bfloat16)
            for s in range(nslots):
                for b in range(B):
                    for hh in range(HG):
                        q_hm[i0 + (s * B + b) * HG + hh] = q4[
                            b, :, s * HG + hh, :]
            qv = q_hm[i0:i0 + nbh].reshape(nbh * 2, 64, DH)
            kv = k_ord[i0:i0 + nbh].reshape(nbh * 2, 64, DH)
            s_ = lax.dot_general(
                qv, kv,
                dimension_numbers=(((2,), (2,)), ((0,), (0,))),
                preferred_element_type=jnp.float32,
            ) * 0.125
            m = jnp.max(s_, axis=-1, keepdims=True)
            w = jnp.exp(s_ - m)
            wsum = jnp.sum(w, axis=-1, keepdims=True)
            w = (w / wsum).astype(jnp.bfloat16)
            ctx = lax.dot_general(
                w, v_ord[i0:i0 + nbh].reshape(nbh * 2, 64, DH),
                dimension_numbers=(((2,), (1,)), ((0,), (0,))),
                preferred_element_type=jnp.float32,
            ).reshape(nbh, SQ_LOC, DH)
            for s in range(nslots):
                for b in range(B):
                    for hh in range(HG):
                        col = (s * HG + hh) * DH
                        ctx_buf[pl.ds(b * SQ_LOC, SQ_LOC), pl.ds(col, DH)] = (
                            ctx[(s * B + b) * HG + hh].astype(jnp.bfloat16))
            part = jnp.dot(ctx_buf[...][:, 0:nc], wo_full[c0:c0 + nc, :],
                           preferred_element_type=jnp.float32)
            if first:
                acc[...] = part
            else:
                acc[...] = acc[...] + part

        p1 = []
        for i in range(3):
            p1 += chunk_copy(0, 1 + i, p1_send, p1_recv, i, nbr[i])

        @pl.when(is_even)
        def _():
            xb[...] = x_ref[...].reshape(B * SQ_LOC, D_MODEL).astype(
                jnp.bfloat16)
            for g in range(N_DEV):
                for b in range(B):
                    for hh in range(HG):
                        k_cm[g, b * HG + hh] = k_ref[
                            b, :, g * HG + hh, :].astype(jnp.bfloat16)
                        v_cm[g, b * HG + hh] = v_ref[
                            b, :, g * HG + hh, :].astype(jnp.bfloat16)
            for s in range(N_DEV):
                o = jnp.bitwise_xor(me, SLOT_XOR[s])
                k_ord[s * B * HG:(s + 1) * B * HG] = k_cm[pl.ds(o, 1)].reshape(
                    B * HG, SKV, DH)
                v_ord[s * B * HG:(s + 1) * B * HG] = v_cm[pl.ds(o, 1)].reshape(
                    B * HG, SKV, DH)

        for r in p1:
            r.wait()

        p2 = []
        for i in range(3):
            p2 += chunk_copy(1 + (i + 1) % 3, 4 + i, p2_send, p2_recv,
                             i, nbr[i])

        @pl.when(is_even)
        def _():
            compute_part(0, 4, first=True)

        for r in p2:
            r.wait()

        p3 = []
        src_slot = (5, 6, 4)
        for i, (kind, a0, a1) in enumerate(
                (("wq", 0, HALF), ("wq", HALF, CHUNK), (None, 0, 0))):
            if kind == "wq":
                rq = pltpu.make_async_remote_copy(
                    src_ref=wq_full.at[:, pl.ds(src_slot[i] * CHUNK + a0,
                                                a1 - a0)],
                    dst_ref=wq_full.at[:, pl.ds(7 * CHUNK + a0, a1 - a0)],
                    send_sem=p3_send.at[i], recv_sem=p3_recv.at[i],
                    device_id=(nbr[i],), device_id_type=pl.DeviceIdType.MESH,
                )
                rq.start()
                p3.append(rq)
            r0, r1 = WO_CUT[i], WO_CUT[i + 1]
            ro = pltpu.make_async_remote_copy(
                src_ref=wo_full.at[pl.ds(src_slot[i] * CHUNK + r0, r1 - r0), :],
                dst_ref=wo_full.at[pl.ds(7 * CHUNK + r0, r1 - r0), :],
                send_sem=p3_send.at[3 + i], recv_sem=p3_recv.at[3 + i],
                device_id=(nbr[i],), device_id_type=pl.DeviceIdType.MESH,
            )
            ro.start()
            p3.append(ro)

        @pl.when(is_even)
        def _():
            compute_part(4, 3, first=False)

        for r in p3:
            r.wait()

        @pl.when(is_even)
        def _():
            compute_part(7, 1, first=False)
            out_ref[...] = acc[...].reshape(B, SQ_LOC, D_MODEL)

        @pl.when(jnp.logical_not(is_even))
        def _():
            out_ref[...] = jnp.zeros((B, SQ_LOC, D_MODEL), jnp.float32)

    return pl.pallas_call(
        body,
        out_shape=jax.ShapeDtypeStruct((B, SQ_LOC, D_MODEL), jnp.float32),
        in_specs=[pl.BlockSpec(memory_space=pltpu.VMEM)] * 5,
        out_specs=pl.BlockSpec(memory_space=pltpu.VMEM),
        scratch_shapes=[
            pltpu.VMEM((D_MODEL, D_FF), jnp.bfloat16),
            pltpu.VMEM((D_FF, D_MODEL), jnp.bfloat16),
            pltpu.VMEM((B * SQ_LOC, D_MODEL), jnp.bfloat16),
            pltpu.VMEM((N_DEV, B * HG, SKV, DH), jnp.bfloat16),
            pltpu.VMEM((N_DEV, B * HG, SKV, DH), jnp.bfloat16),
            pltpu.VMEM((N_DEV * B * HG, SKV, DH), jnp.bfloat16),
            pltpu.VMEM((N_DEV * B * HG, SKV, DH), jnp.bfloat16),
            pltpu.VMEM((N_DEV * B * HG, SQ_LOC, DH), jnp.bfloat16),
            pltpu.VMEM((B * SQ_LOC, 4 * CHUNK), jnp.bfloat16),
            pltpu.VMEM((B * SQ_LOC, D_MODEL), jnp.float32),
            pltpu.SemaphoreType.DMA((2, 3)),
            pltpu.SemaphoreType.DMA((2, 3)),
            pltpu.SemaphoreType.DMA((2, 3)),
            pltpu.SemaphoreType.DMA((2, 3)),
            pltpu.SemaphoreType.DMA((6,)),
            pltpu.SemaphoreType.DMA((6,)),
        ],
        compiler_params=pltpu.CompilerParams(collective_id=0),
    )(x, Wq, K_ext, V_ext, Wo)
